# Initial kernel scaffold; baseline (speedup 1.0000x reference)
import jax
import jax.numpy as jnp
from jax.experimental import pallas as pl


def kernel(query, key, value, reference_points_cam, bev_mask, spatial_shapes, level_start_index, W_off, b_off, W_attn, b_attn, W_val, b_val, W_out, b_out):
    raise NotImplementedError("write your pallas kernel here")


# speedup vs baseline: 42.4616x; 42.4616x over previous
import functools

import jax
import jax.numpy as jnp
from jax import lax
from jax.experimental import pallas as pl
from jax.experimental.pallas import tpu as pltpu
from jax.experimental.pallas import tpu_sc as plsc

EMBED = 256
HEADS = 8
LEVELS = 4
POINTS = 8
CAMS = 6
NQ = 2500
D = 4
SHAPES = ((50, 80), (25, 40), (13, 20), (7, 10))
LSTART = (0, 4000, 5000, 5260)
L = 5330
HD = EMBED // HEADS

QP = 2560
QB = 128
NW = 32
QPW = QP // NW
IDX_PER_Q = CAMS * HEADS * LEVELS * POINTS
WGT_PER_Q = IDX_PER_Q * 4
CHUNK = LEVELS * POINTS
NCHUNK = CAMS * HEADS
SEG = tuple(h * w + w + 1 for h, w in SHAPES)
LOFF = (0, SEG[0], SEG[0] + SEG[1], SEG[0] + SEG[1] + SEG[2])
PLANE = sum(SEG)


def _vproj_body(x_ref, w_ref, b_ref, o_ref):
    o_ref[...] = jnp.dot(x_ref[...], w_ref[...],
                         preferred_element_type=jnp.float32) + b_ref[...]


def _vproj(x, w, b):
    n = x.shape[0]
    blk = 512
    grid = (n + blk - 1) // blk
    return pl.pallas_call(
        _vproj_body,
        grid=(grid,),
        in_specs=[
            pl.BlockSpec((blk, EMBED), lambda i: (i, 0)),
            pl.BlockSpec((EMBED, EMBED), lambda i: (0, 0)),
            pl.BlockSpec((1, EMBED), lambda i: (0, 0)),
        ],
        out_specs=pl.BlockSpec((blk, EMBED), lambda i: (i, 0)),
        out_shape=jax.ShapeDtypeStruct((n, EMBED), jnp.float32),
    )(x, w, b)


def _expand_body(qT_ref, refT_ref, mk_ref, woff_ref, boff_ref, wattn_ref,
                 battn_ref, base_ref, idx_ref, wgt_ref, m_ref):
    q = qT_ref[...]
    off = jnp.dot(woff_ref[...], q,
                  preferred_element_type=jnp.float32) + boff_ref[...]
    logits = jnp.dot(wattn_ref[...], q,
                     preferred_element_type=jnp.float32) + battn_ref[...]
    aw = []
    for h in range(HEADS):
        sl = logits[h * 32:(h + 1) * 32, :]
        mx = jnp.max(sl, axis=0, keepdims=True)
        e = jnp.exp(sl - mx)
        aw.append(e / jnp.sum(e, axis=0, keepdims=True))
    mk = mk_ref[...]
    base0 = base_ref[...]
    mrows = []
    for cam in range(CAMS):
        msum = jnp.sum(mk[cam * D:(cam + 1) * D, :], axis=0, keepdims=True)
        mcam = (msum > 0.0).astype(jnp.float32)
        mrows.append(mcam)
        refx = refT_ref[cam * D:(cam + 1) * D, :]
        refy = refT_ref[24 + cam * D:24 + (cam + 1) * D, :]
        refx_t = jnp.concatenate([refx] * 16, axis=0)
        refy_t = jnp.concatenate([refy] * 16, axis=0)
        for lvl in range(LEVELS):
            Hl, Wl = SHAPES[lvl]
            x = refx_t * Wl + off[lvl * 64:(lvl + 1) * 64, :] - 0.5
            y = refy_t * Hl + off[256 + lvl * 64:256 + (lvl + 1) * 64, :] - 0.5
            x0 = jnp.floor(x)
            y0 = jnp.floor(y)
            wx1 = x - x0
            wy1 = y - y0
            aw_l = jnp.concatenate(
                [aw[h][lvl * POINTS:(lvl + 1) * POINTS, :]
                 for h in range(HEADS)], axis=0)
            wbase = aw_l * mcam
            basei = base0 + (cam * HEADS * PLANE + LOFF[lvl] + Wl + 1)
            x0c = jnp.clip(x0, -1.0, Wl - 1).astype(jnp.int32)
            y0c = jnp.clip(y0, -1.0, Hl - 1).astype(jnp.int32)
            idx = basei + y0c * Wl + x0c
            ri = (cam * LEVELS + lvl) * 64
            idx_ref[ri:ri + 64, :] = idx
            for tap, (dy, dx) in enumerate(((0, 0), (0, 1), (1, 0), (1, 1))):
                ixf = x0 + dx
                iyf = y0 + dy
                valid = ((ixf >= 0.0) & (ixf <= Wl - 1)
                         & (iyf >= 0.0) & (iyf <= Hl - 1))
                wt = wbase * (wx1 if dx else 1.0 - wx1) \
                    * (wy1 if dy else 1.0 - wy1) * valid.astype(jnp.float32)
                r0 = ((cam * LEVELS + lvl) * 4 + tap) * 64
                wgt_ref[r0:r0 + 64, :] = wt
    m_ref[...] = jnp.concatenate(mrows, axis=0)


def _expand(qT, refT, maskf, woffp, boffp, wattnT, battn, base_row):
    grid = QP // QB
    return pl.pallas_call(
        _expand_body,
        grid=(grid,),
        in_specs=[
            pl.BlockSpec((EMBED, QB), lambda i: (0, i)),
            pl.BlockSpec((48, QB), lambda i: (0, i)),
            pl.BlockSpec((24, QB), lambda i: (0, i)),
            pl.BlockSpec((512, EMBED), lambda i: (0, 0)),
            pl.BlockSpec((512, 1), lambda i: (0, 0)),
            pl.BlockSpec((EMBED, EMBED), lambda i: (0, 0)),
            pl.BlockSpec((EMBED, 1), lambda i: (0, 0)),
            pl.BlockSpec((64, QB), lambda i: (0, 0)),
        ],
        out_specs=[
            pl.BlockSpec((IDX_PER_Q, QB), lambda i: (0, i)),
            pl.BlockSpec((WGT_PER_Q, QB), lambda i: (0, i)),
            pl.BlockSpec((CAMS, QB), lambda i: (0, i)),
        ],
        out_shape=[
            jax.ShapeDtypeStruct((IDX_PER_Q, QP), jnp.int32),
            jax.ShapeDtypeStruct((WGT_PER_Q, QP), jnp.float32),
            jax.ShapeDtypeStruct((CAMS, QP), jnp.float32),
        ],
    )(qT, refT, maskf, woffp, boffp, wattnT, battn, base_row)


def _sc_gather(v_t, idx_q, wgt_q):
    mesh = plsc.VectorSubcoreMesh(core_axis_name="c", subcore_axis_name="s")

    @functools.partial(
        pl.kernel,
        mesh=mesh,
        out_type=jax.ShapeDtypeStruct((QP, EMBED), jnp.float32),
        scratch_types=[
            pltpu.VMEM((IDX_PER_Q,), jnp.int32),
            pltpu.VMEM((WGT_PER_Q,), jnp.float32),
            pltpu.VMEM((CHUNK, 128), jnp.float32),
            pltpu.VMEM((CHUNK, 128), jnp.float32),
            pltpu.VMEM((EMBED,), jnp.float32),
            pltpu.SemaphoreType.DMA,
            pltpu.SemaphoreType.DMA,
        ],
    )
    def body(v_hbm, idx_hbm, wgt_hbm, out_hbm, idx_v, wgt_v, buf0, buf1,
             out_v, sem0, sem1):
        wid = lax.axis_index("s") * 2 + lax.axis_index("c")
        q0 = wid * QPW
        bufs = (buf0, buf1)
        sems = (sem0, sem1)
        zero = jnp.zeros((16,), jnp.float32)

        def process_chunk(c, buf):
            woff = c * CHUNK * 4

            def mac(i, accs):
                a = list(accs)
                r0 = i * 4
                wv = wgt_v[pl.ds(woff + r0 * 4, 16)]
                for k in range(4):
                    for t in range(4):
                        w = wv[k * 4 + t]
                        a[2 * t] = a[2 * t] \
                            + w * buf[r0 + k, pl.ds(32 * t, 16)]
                        a[2 * t + 1] = a[2 * t + 1] \
                            + w * buf[r0 + k, pl.ds(32 * t + 16, 16)]
                return tuple(a)

            accs = lax.fori_loop(0, CHUNK // 4, mac, (zero,) * 8)
            acc0 = accs[0] + accs[2] + accs[4] + accs[6]
            acc1 = accs[1] + accs[3] + accs[5] + accs[7]
            h = c % HEADS
            s0 = h * HD
            if c < HEADS:
                out_v[pl.ds(s0, 16)] = acc0
                out_v[pl.ds(s0 + 16, 16)] = acc1
            else:
                out_v[pl.ds(s0, 16)] = out_v[pl.ds(s0, 16)] + acc0
                out_v[pl.ds(s0 + 16, 16)] = out_v[pl.ds(s0 + 16, 16)] + acc1

        def per_q(q, carry):
            gq = q0 + q
            pltpu.sync_copy(idx_hbm.at[gq], idx_v)
            pltpu.sync_copy(wgt_hbm.at[gq], wgt_v)
            handles = []
            for c in range(NCHUNK):
                handles.append(pltpu.async_copy(
                    v_hbm.at[idx_v.at[pl.ds(c * CHUNK, CHUNK)]],
                    bufs[c % 2], sems[c % 2]))
                if c >= 1:
                    handles[c - 1].wait()
                    process_chunk(c - 1, bufs[(c - 1) % 2])
            handles[NCHUNK - 1].wait()
            process_chunk(NCHUNK - 1, bufs[(NCHUNK - 1) % 2])
            pltpu.sync_copy(out_v, out_hbm.at[gq])
            return carry

        lax.fori_loop(0, QPW, per_q, 0)

    return body(v_t, idx_q, wgt_q)


def _outproj_body(s_ref, m_ref, q_ref, w_ref, b_ref, o_ref):
    cnt = jnp.maximum(jnp.sum(m_ref[...], axis=1, keepdims=True), 1.0)
    s = s_ref[...] / cnt
    o_ref[...] = jnp.dot(s, w_ref[...],
                         preferred_element_type=jnp.float32) \
        + b_ref[...] + q_ref[...]


def _outproj(slots, mq, qpad, w, b):
    blk = 512
    grid = QP // blk
    return pl.pallas_call(
        _outproj_body,
        grid=(grid,),
        in_specs=[
            pl.BlockSpec((blk, EMBED), lambda i: (i, 0)),
            pl.BlockSpec((blk, CAMS), lambda i: (i, 0)),
            pl.BlockSpec((blk, EMBED), lambda i: (i, 0)),
            pl.BlockSpec((EMBED, EMBED), lambda i: (0, 0)),
            pl.BlockSpec((1, EMBED), lambda i: (0, 0)),
        ],
        out_specs=pl.BlockSpec((blk, EMBED), lambda i: (i, 0)),
        out_shape=jax.ShapeDtypeStruct((QP, EMBED), jnp.float32),
    )(slots, mq, qpad, w, b)


def kernel(query, key, value, reference_points_cam, bev_mask, spatial_shapes,
           level_start_index, W_off, b_off, W_attn, b_attn, W_val, b_val,
           W_out, b_out):
    q2 = query[0]
    v3 = value[:, :, 0, :].reshape(CAMS * L, EMBED)

    vproj = _vproj(v3, W_val, b_val.reshape(1, EMBED))
    v5 = vproj.reshape(CAMS, L, HEADS, HD).transpose(0, 2, 1, 3)
    vp = jnp.pad(v5, ((0, 0), (0, 0), (96, 96), (0, 0)))
    segs = []
    for lvl, (Hl, Wl) in enumerate(SHAPES):
        start = 96 + LSTART[lvl] - (Wl + 1)
        segs.append(jnp.concatenate(
            [vp[:, :, start + sh:start + sh + SEG[lvl], :]
             for sh in (0, 1, Wl, Wl + 1)], axis=-1))
    v4 = jnp.concatenate(segs, axis=2) \
        .reshape(CAMS * HEADS * PLANE, 4 * HD)

    pad = QP - NQ
    qT = jnp.pad(q2.T, ((0, 0), (0, pad)))
    rpc = reference_points_cam[:, 0]
    refT = jnp.pad(rpc.transpose(3, 0, 2, 1).reshape(48, NQ),
                   ((0, 0), (0, pad)))
    maskf = jnp.pad(
        bev_mask[:, 0].astype(jnp.float32).transpose(0, 2, 1).reshape(24, NQ),
        ((0, 0), (0, pad)))
    woffp = W_off.reshape(EMBED, HEADS, LEVELS, POINTS, 2) \
        .transpose(4, 2, 1, 3, 0).reshape(512, EMBED)
    boffp = b_off.reshape(HEADS, LEVELS, POINTS, 2) \
        .transpose(3, 1, 0, 2).reshape(512, 1)
    wattnT = W_attn.T
    battn = b_attn.reshape(EMBED, 1)
    base_row = jnp.broadcast_to(
        (jnp.repeat(jnp.arange(HEADS, dtype=jnp.int32), POINTS) * PLANE)
        .reshape(64, 1), (64, QB))

    idxT, wgtT, mT = _expand(qT, refT, maskf, woffp, boffp, wattnT, battn,
                             base_row)
    idx_q = idxT.reshape(CAMS, LEVELS, HEADS, 2, D, QP) \
        .transpose(5, 0, 2, 1, 3, 4).reshape(QP, IDX_PER_Q)
    wgt_q = wgtT.reshape(CAMS, LEVELS, 4, HEADS, 2, D, QP) \
        .transpose(6, 0, 3, 1, 4, 5, 2).reshape(QP, WGT_PER_Q)

    slots = _sc_gather(v4, idx_q, wgt_q)

    out = _outproj(slots, mT.T, jnp.pad(q2, ((0, pad), (0, 0))),
                   W_out, b_out.reshape(1, EMBED))
    return out[:NQ].reshape(1, NQ, EMBED)


# speedup vs baseline: 58.2741x; 1.3724x over previous
import functools

import jax
import jax.numpy as jnp
from jax import lax
from jax.experimental import pallas as pl
from jax.experimental.pallas import tpu as pltpu
from jax.experimental.pallas import tpu_sc as plsc

EMBED = 256
HEADS = 8
LEVELS = 4
POINTS = 8
CAMS = 6
NQ = 2500
D = 4
SHAPES = ((50, 80), (25, 40), (13, 20), (7, 10))
LSTART = (0, 4000, 5000, 5260)
L = 5330
HD = EMBED // HEADS

QP = 2560
QB = 128
NW = 32
QPW = QP // NW
IDX_PER_Q = CAMS * HEADS * LEVELS * POINTS
WGT_PER_Q = IDX_PER_Q * 4
CHUNK = LEVELS * POINTS
NCHUNK = CAMS * HEADS
SEG = tuple(h * w + w + 1 for h, w in SHAPES)
LOFF = (0, SEG[0], SEG[0] + SEG[1], SEG[0] + SEG[1] + SEG[2])
PLANE = sum(SEG)


def _vproj_body(x_ref, w_ref, b_ref, o_ref):
    o_ref[...] = jnp.dot(x_ref[...], w_ref[...],
                         preferred_element_type=jnp.float32) + b_ref[...]


def _vproj(x, w, b):
    n = x.shape[0]
    blk = 512
    grid = (n + blk - 1) // blk
    return pl.pallas_call(
        _vproj_body,
        grid=(grid,),
        in_specs=[
            pl.BlockSpec((blk, EMBED), lambda i: (i, 0)),
            pl.BlockSpec((EMBED, EMBED), lambda i: (0, 0)),
            pl.BlockSpec((1, EMBED), lambda i: (0, 0)),
        ],
        out_specs=pl.BlockSpec((blk, EMBED), lambda i: (i, 0)),
        out_shape=jax.ShapeDtypeStruct((n, EMBED), jnp.float32),
    )(x, w, b)


def _expand_body(qT_ref, refT_ref, mk_ref, woff_ref, boff_ref, wattn_ref,
                 battn_ref, base_ref, idx_ref, wgt_ref, m_ref):
    q = qT_ref[...]
    off = jnp.dot(woff_ref[...], q,
                  preferred_element_type=jnp.float32) + boff_ref[...]
    logits = jnp.dot(wattn_ref[...], q,
                     preferred_element_type=jnp.float32) + battn_ref[...]
    aw = []
    for h in range(HEADS):
        sl = logits[h * 32:(h + 1) * 32, :]
        mx = jnp.max(sl, axis=0, keepdims=True)
        e = jnp.exp(sl - mx)
        aw.append(e / jnp.sum(e, axis=0, keepdims=True))
    mk = mk_ref[...]
    base0 = base_ref[...]
    mrows = []
    for cam in range(CAMS):
        msum = jnp.sum(mk[cam * D:(cam + 1) * D, :], axis=0, keepdims=True)
        mcam = (msum > 0.0).astype(jnp.float32)
        mrows.append(mcam)
        refx = refT_ref[cam * D:(cam + 1) * D, :]
        refy = refT_ref[24 + cam * D:24 + (cam + 1) * D, :]
        refx_t = jnp.concatenate([refx] * 16, axis=0)
        refy_t = jnp.concatenate([refy] * 16, axis=0)
        for lvl in range(LEVELS):
            Hl, Wl = SHAPES[lvl]
            x = refx_t * Wl + off[lvl * 64:(lvl + 1) * 64, :] - 0.5
            y = refy_t * Hl + off[256 + lvl * 64:256 + (lvl + 1) * 64, :] - 0.5
            x0 = jnp.floor(x)
            y0 = jnp.floor(y)
            wx1 = x - x0
            wy1 = y - y0
            aw_l = jnp.concatenate(
                [aw[h][lvl * POINTS:(lvl + 1) * POINTS, :]
                 for h in range(HEADS)], axis=0)
            wbase = aw_l * mcam
            basei = base0 + (cam * HEADS * PLANE + LOFF[lvl] + Wl + 1)
            x0c = jnp.clip(x0, -1.0, Wl - 1).astype(jnp.int32)
            y0c = jnp.clip(y0, -1.0, Hl - 1).astype(jnp.int32)
            idx = basei + y0c * Wl + x0c
            ri = (cam * LEVELS + lvl) * 64
            idx_ref[ri:ri + 64, :] = idx
            for tap, (dy, dx) in enumerate(((0, 0), (0, 1), (1, 0), (1, 1))):
                ixf = x0 + dx
                iyf = y0 + dy
                valid = ((ixf >= 0.0) & (ixf <= Wl - 1)
                         & (iyf >= 0.0) & (iyf <= Hl - 1))
                wt = wbase * (wx1 if dx else 1.0 - wx1) \
                    * (wy1 if dy else 1.0 - wy1) * valid.astype(jnp.float32)
                r0 = ((cam * LEVELS + lvl) * 4 + tap) * 64
                wgt_ref[r0:r0 + 64, :] = wt
    m_ref[...] = jnp.concatenate(mrows, axis=0)


def _expand(qT, refT, maskf, woffp, boffp, wattnT, battn, base_row):
    grid = QP // QB
    return pl.pallas_call(
        _expand_body,
        grid=(grid,),
        in_specs=[
            pl.BlockSpec((EMBED, QB), lambda i: (0, i)),
            pl.BlockSpec((48, QB), lambda i: (0, i)),
            pl.BlockSpec((24, QB), lambda i: (0, i)),
            pl.BlockSpec((512, EMBED), lambda i: (0, 0)),
            pl.BlockSpec((512, 1), lambda i: (0, 0)),
            pl.BlockSpec((EMBED, EMBED), lambda i: (0, 0)),
            pl.BlockSpec((EMBED, 1), lambda i: (0, 0)),
            pl.BlockSpec((64, QB), lambda i: (0, 0)),
        ],
        out_specs=[
            pl.BlockSpec((IDX_PER_Q, QB), lambda i: (0, i)),
            pl.BlockSpec((WGT_PER_Q, QB), lambda i: (0, i)),
            pl.BlockSpec((CAMS, QB), lambda i: (0, i)),
        ],
        out_shape=[
            jax.ShapeDtypeStruct((IDX_PER_Q, QP), jnp.int32),
            jax.ShapeDtypeStruct((WGT_PER_Q, QP), jnp.float32),
            jax.ShapeDtypeStruct((CAMS, QP), jnp.float32),
        ],
    )(qT, refT, maskf, woffp, boffp, wattnT, battn, base_row)


def _sc_gather(v_t, idx_q, wgt_q):
    mesh = plsc.VectorSubcoreMesh(core_axis_name="c", subcore_axis_name="s")
    BIG = 128
    NBIG = IDX_PER_Q // BIG

    @functools.partial(
        pl.kernel,
        mesh=mesh,
        out_type=jax.ShapeDtypeStruct((QP, EMBED), jnp.float32),
        scratch_types=[
            pltpu.VMEM((IDX_PER_Q,), jnp.int32),
            pltpu.VMEM((WGT_PER_Q,), jnp.float32),
            pltpu.VMEM((BIG, 128), jnp.float32),
            pltpu.VMEM((BIG, 128), jnp.float32),
            pltpu.VMEM((EMBED,), jnp.float32),
            pltpu.SemaphoreType.DMA,
            pltpu.SemaphoreType.DMA,
        ],
    )
    def body(v_hbm, idx_hbm, wgt_hbm, out_hbm, idx_v, wgt_v, buf0, buf1,
             out_v, sem0, sem1):
        wid = lax.axis_index("s") * 2 + lax.axis_index("c")
        q0 = wid * QPW
        bufs = (buf0, buf1)
        sems = (sem0, sem1)
        zero = jnp.zeros((16,), jnp.float32)

        def process_big(bc, buf):
            for s in range(BIG // CHUNK):
                c = bc * (BIG // CHUNK) + s
                woff = c * CHUNK * 4
                roff = s * CHUNK

                def mac(i, accs):
                    a = list(accs)
                    r0 = i * 4
                    wv = wgt_v[pl.ds(woff + r0 * 4, 16)]
                    for k in range(4):
                        for t in range(4):
                            w = wv[k * 4 + t]
                            a[2 * t] = a[2 * t] \
                                + w * buf[roff + r0 + k, pl.ds(32 * t, 16)]
                            a[2 * t + 1] = a[2 * t + 1] \
                                + w * buf[roff + r0 + k,
                                          pl.ds(32 * t + 16, 16)]
                    return tuple(a)

                accs = lax.fori_loop(0, CHUNK // 4, mac, (zero,) * 8)
                acc0 = accs[0] + accs[2] + accs[4] + accs[6]
                acc1 = accs[1] + accs[3] + accs[5] + accs[7]
                h = c % HEADS
                s0 = h * HD
                if c < HEADS:
                    out_v[pl.ds(s0, 16)] = acc0
                    out_v[pl.ds(s0 + 16, 16)] = acc1
                else:
                    out_v[pl.ds(s0, 16)] = out_v[pl.ds(s0, 16)] + acc0
                    out_v[pl.ds(s0 + 16, 16)] = \
                        out_v[pl.ds(s0 + 16, 16)] + acc1

        def per_q(q, carry):
            gq = q0 + q
            pltpu.sync_copy(idx_hbm.at[gq], idx_v)
            pltpu.sync_copy(wgt_hbm.at[gq], wgt_v)
            handles = []
            for bc in range(NBIG):
                handles.append(pltpu.async_copy(
                    v_hbm.at[idx_v.at[pl.ds(bc * BIG, BIG)]],
                    bufs[bc % 2], sems[bc % 2]))
                if bc >= 1:
                    handles[bc - 1].wait()
                    process_big(bc - 1, bufs[(bc - 1) % 2])
            handles[NBIG - 1].wait()
            process_big(NBIG - 1, bufs[(NBIG - 1) % 2])
            pltpu.sync_copy(out_v, out_hbm.at[gq])
            return carry

        lax.fori_loop(0, QPW, per_q, 0)

    return body(v_t, idx_q, wgt_q)


def _outproj_body(s_ref, m_ref, q_ref, w_ref, b_ref, o_ref):
    cnt = jnp.maximum(jnp.sum(m_ref[...], axis=1, keepdims=True), 1.0)
    s = s_ref[...] / cnt
    o_ref[...] = jnp.dot(s, w_ref[...],
                         preferred_element_type=jnp.float32) \
        + b_ref[...] + q_ref[...]


def _outproj(slots, mq, qpad, w, b):
    blk = 512
    grid = QP // blk
    return pl.pallas_call(
        _outproj_body,
        grid=(grid,),
        in_specs=[
            pl.BlockSpec((blk, EMBED), lambda i: (i, 0)),
            pl.BlockSpec((blk, CAMS), lambda i: (i, 0)),
            pl.BlockSpec((blk, EMBED), lambda i: (i, 0)),
            pl.BlockSpec((EMBED, EMBED), lambda i: (0, 0)),
            pl.BlockSpec((1, EMBED), lambda i: (0, 0)),
        ],
        out_specs=pl.BlockSpec((blk, EMBED), lambda i: (i, 0)),
        out_shape=jax.ShapeDtypeStruct((QP, EMBED), jnp.float32),
    )(slots, mq, qpad, w, b)


def kernel(query, key, value, reference_points_cam, bev_mask, spatial_shapes,
           level_start_index, W_off, b_off, W_attn, b_attn, W_val, b_val,
           W_out, b_out):
    q2 = query[0]
    v3 = value[:, :, 0, :].reshape(CAMS * L, EMBED)

    vproj = _vproj(v3, W_val, b_val.reshape(1, EMBED))
    v5 = vproj.reshape(CAMS, L, HEADS, HD).transpose(0, 2, 1, 3)
    vp = jnp.pad(v5, ((0, 0), (0, 0), (96, 96), (0, 0)))
    segs = []
    for lvl, (Hl, Wl) in enumerate(SHAPES):
        start = 96 + LSTART[lvl] - (Wl + 1)
        segs.append(jnp.concatenate(
            [vp[:, :, start + sh:start + sh + SEG[lvl], :]
             for sh in (0, 1, Wl, Wl + 1)], axis=-1))
    v4 = jnp.concatenate(segs, axis=2) \
        .reshape(CAMS * HEADS * PLANE, 4 * HD)

    pad = QP - NQ
    qT = jnp.pad(q2.T, ((0, 0), (0, pad)))
    rpc = reference_points_cam[:, 0]
    refT = jnp.pad(rpc.transpose(3, 0, 2, 1).reshape(48, NQ),
                   ((0, 0), (0, pad)))
    maskf = jnp.pad(
        bev_mask[:, 0].astype(jnp.float32).transpose(0, 2, 1).reshape(24, NQ),
        ((0, 0), (0, pad)))
    woffp = W_off.reshape(EMBED, HEADS, LEVELS, POINTS, 2) \
        .transpose(4, 2, 1, 3, 0).reshape(512, EMBED)
    boffp = b_off.reshape(HEADS, LEVELS, POINTS, 2) \
        .transpose(3, 1, 0, 2).reshape(512, 1)
    wattnT = W_attn.T
    battn = b_attn.reshape(EMBED, 1)
    base_row = jnp.broadcast_to(
        (jnp.repeat(jnp.arange(HEADS, dtype=jnp.int32), POINTS) * PLANE)
        .reshape(64, 1), (64, QB))

    idxT, wgtT, mT = _expand(qT, refT, maskf, woffp, boffp, wattnT, battn,
                             base_row)
    idx_q = idxT.reshape(CAMS, LEVELS, HEADS, 2, D, QP) \
        .transpose(5, 0, 2, 1, 3, 4).reshape(QP, IDX_PER_Q)
    wgt_q = wgtT.reshape(CAMS, LEVELS, 4, HEADS, 2, D, QP) \
        .transpose(6, 0, 3, 1, 4, 5, 2).reshape(QP, WGT_PER_Q)

    slots = _sc_gather(v4, idx_q, wgt_q)

    out = _outproj(slots, mT.T, jnp.pad(q2, ((0, pad), (0, 0))),
                   W_out, b_out.reshape(1, EMBED))
    return out[:NQ].reshape(1, NQ, EMBED)


# speedup vs baseline: 63.1825x; 1.0842x over previous
import functools

import jax
import jax.numpy as jnp
from jax import lax
from jax.experimental import pallas as pl
from jax.experimental.pallas import tpu as pltpu
from jax.experimental.pallas import tpu_sc as plsc

EMBED = 256
HEADS = 8
LEVELS = 4
POINTS = 8
CAMS = 6
NQ = 2500
D = 4
SHAPES = ((50, 80), (25, 40), (13, 20), (7, 10))
LSTART = (0, 4000, 5000, 5260)
L = 5330
HD = EMBED // HEADS

QP = 2560
QB = 128
NW = 32
QPW = QP // NW
IDX_PER_Q = CAMS * HEADS * LEVELS * POINTS
WGT_PER_Q = IDX_PER_Q * 4
CHUNK = LEVELS * POINTS
NCHUNK = CAMS * HEADS
SEG = tuple(h * w + w + 1 for h, w in SHAPES)
LOFF = (0, SEG[0], SEG[0] + SEG[1], SEG[0] + SEG[1] + SEG[2])
PLANE = sum(SEG)


def _vproj_body(x_ref, w_ref, b_ref, o_ref):
    o_ref[...] = jnp.dot(x_ref[...], w_ref[...],
                         preferred_element_type=jnp.float32) + b_ref[...]


def _vproj(x, w, b):
    n = x.shape[0]
    blk = 512
    grid = (n + blk - 1) // blk
    return pl.pallas_call(
        _vproj_body,
        grid=(grid,),
        in_specs=[
            pl.BlockSpec((blk, EMBED), lambda i: (i, 0)),
            pl.BlockSpec((EMBED, EMBED), lambda i: (0, 0)),
            pl.BlockSpec((1, EMBED), lambda i: (0, 0)),
        ],
        out_specs=pl.BlockSpec((blk, EMBED), lambda i: (i, 0)),
        out_shape=jax.ShapeDtypeStruct((n, EMBED), jnp.float32),
    )(x, w, b)


def _expand_body(qT_ref, refT_ref, mk_ref, woff_ref, boff_ref, wattn_ref,
                 battn_ref, base_ref, idx_ref, wgt_ref, m_ref):
    q = qT_ref[...]
    off = jnp.dot(woff_ref[...], q,
                  preferred_element_type=jnp.float32) + boff_ref[...]
    logits = jnp.dot(wattn_ref[...], q,
                     preferred_element_type=jnp.float32) + battn_ref[...]
    aw = []
    for h in range(HEADS):
        sl = logits[h * 32:(h + 1) * 32, :]
        mx = jnp.max(sl, axis=0, keepdims=True)
        e = jnp.exp(sl - mx)
        aw.append(e / jnp.sum(e, axis=0, keepdims=True))
    mk = mk_ref[...]
    base0 = base_ref[...]
    mrows = []
    for cam in range(CAMS):
        msum = jnp.sum(mk[cam * D:(cam + 1) * D, :], axis=0, keepdims=True)
        mcam = (msum > 0.0).astype(jnp.float32)
        mrows.append(mcam)
        refx = refT_ref[cam * D:(cam + 1) * D, :]
        refy = refT_ref[24 + cam * D:24 + (cam + 1) * D, :]
        refx_t = jnp.concatenate([refx] * 16, axis=0)
        refy_t = jnp.concatenate([refy] * 16, axis=0)
        for lvl in range(LEVELS):
            Hl, Wl = SHAPES[lvl]
            x = refx_t * Wl + off[lvl * 64:(lvl + 1) * 64, :] - 0.5
            y = refy_t * Hl + off[256 + lvl * 64:256 + (lvl + 1) * 64, :] - 0.5
            x0 = jnp.floor(x)
            y0 = jnp.floor(y)
            wx1 = x - x0
            wy1 = y - y0
            aw_l = jnp.concatenate(
                [aw[h][lvl * POINTS:(lvl + 1) * POINTS, :]
                 for h in range(HEADS)], axis=0)
            wbase = aw_l * mcam
            basei = base0 + (cam * HEADS * PLANE + LOFF[lvl] + Wl + 1)
            x0c = jnp.clip(x0, -1.0, Wl - 1).astype(jnp.int32)
            y0c = jnp.clip(y0, -1.0, Hl - 1).astype(jnp.int32)
            idx = basei + y0c * Wl + x0c
            ri = (cam * LEVELS + lvl) * 64
            idx_ref[ri:ri + 64, :] = idx
            for tap, (dy, dx) in enumerate(((0, 0), (0, 1), (1, 0), (1, 1))):
                ixf = x0 + dx
                iyf = y0 + dy
                valid = ((ixf >= 0.0) & (ixf <= Wl - 1)
                         & (iyf >= 0.0) & (iyf <= Hl - 1))
                wt = wbase * (wx1 if dx else 1.0 - wx1) \
                    * (wy1 if dy else 1.0 - wy1) * valid.astype(jnp.float32)
                r0 = ((cam * LEVELS + lvl) * 4 + tap) * 64
                wgt_ref[r0:r0 + 64, :] = wt
    m_ref[...] = jnp.concatenate(mrows, axis=0)


def _expand(qT, refT, maskf, woffp, boffp, wattnT, battn, base_row):
    grid = QP // QB
    return pl.pallas_call(
        _expand_body,
        grid=(grid,),
        in_specs=[
            pl.BlockSpec((EMBED, QB), lambda i: (0, i)),
            pl.BlockSpec((48, QB), lambda i: (0, i)),
            pl.BlockSpec((24, QB), lambda i: (0, i)),
            pl.BlockSpec((512, EMBED), lambda i: (0, 0)),
            pl.BlockSpec((512, 1), lambda i: (0, 0)),
            pl.BlockSpec((EMBED, EMBED), lambda i: (0, 0)),
            pl.BlockSpec((EMBED, 1), lambda i: (0, 0)),
            pl.BlockSpec((64, QB), lambda i: (0, 0)),
        ],
        out_specs=[
            pl.BlockSpec((IDX_PER_Q, QB), lambda i: (0, i)),
            pl.BlockSpec((WGT_PER_Q, QB), lambda i: (0, i)),
            pl.BlockSpec((CAMS, QB), lambda i: (0, i)),
        ],
        out_shape=[
            jax.ShapeDtypeStruct((IDX_PER_Q, QP), jnp.int32),
            jax.ShapeDtypeStruct((WGT_PER_Q, QP), jnp.float32),
            jax.ShapeDtypeStruct((CAMS, QP), jnp.float32),
        ],
    )(qT, refT, maskf, woffp, boffp, wattnT, battn, base_row)


def _sc_gather(v_t, idx_q, wgt_q):
    mesh = plsc.VectorSubcoreMesh(core_axis_name="c", subcore_axis_name="s")
    BIG = 128
    NBIG = IDX_PER_Q // BIG
    DEPTH = 4

    @functools.partial(
        pl.kernel,
        mesh=mesh,
        out_type=jax.ShapeDtypeStruct((QP, EMBED), jnp.float32),
        scratch_types=(
            [pltpu.VMEM((IDX_PER_Q,), jnp.int32)] * 2
            + [pltpu.VMEM((WGT_PER_Q,), jnp.float32)] * 2
            + [pltpu.VMEM((BIG, 128), jnp.float32)] * DEPTH
            + [pltpu.VMEM((EMBED,), jnp.float32)]
            + [pltpu.SemaphoreType.DMA] * (DEPTH + 2)
        ),
    )
    def body(v_hbm, idx_hbm, wgt_hbm, out_hbm, idx_v0, idx_v1, wgt_v0,
             wgt_v1, b0, b1, b2, b3, out_v, g0, g1, g2, g3, si0, si1):
        wid = lax.axis_index("s") * 2 + lax.axis_index("c")
        q0 = wid * QPW
        bufs = (b0, b1, b2, b3)
        gsems = (g0, g1, g2, g3)
        isets = ((idx_v0, wgt_v0, si0), (idx_v1, wgt_v1, si1))
        zero = jnp.zeros((16,), jnp.float32)

        def process_big(wgt_v, bc, buf):
            for s in range(BIG // CHUNK):
                c = bc * (BIG // CHUNK) + s
                woff = c * CHUNK * 4
                roff = s * CHUNK

                def mac(i, accs):
                    a = list(accs)
                    r0 = i * 4
                    wv = wgt_v[pl.ds(woff + r0 * 4, 16)]
                    for k in range(4):
                        for t in range(4):
                            w = wv[k * 4 + t]
                            a[2 * t] = a[2 * t] \
                                + w * buf[roff + r0 + k, pl.ds(32 * t, 16)]
                            a[2 * t + 1] = a[2 * t + 1] \
                                + w * buf[roff + r0 + k,
                                          pl.ds(32 * t + 16, 16)]
                    return tuple(a)

                accs = lax.fori_loop(0, CHUNK // 4, mac, (zero,) * 8)
                acc0 = accs[0] + accs[2] + accs[4] + accs[6]
                acc1 = accs[1] + accs[3] + accs[5] + accs[7]
                h = c % HEADS
                s0 = h * HD
                if c < HEADS:
                    out_v[pl.ds(s0, 16)] = acc0
                    out_v[pl.ds(s0 + 16, 16)] = acc1
                else:
                    out_v[pl.ds(s0, 16)] = out_v[pl.ds(s0, 16)] + acc0
                    out_v[pl.ds(s0 + 16, 16)] = \
                        out_v[pl.ds(s0 + 16, 16)] + acc1

        def fetch(gq, iset):
            idx_v, wgt_v, sem = iset
            pltpu.async_copy(idx_hbm.at[gq], idx_v, sem)
            pltpu.async_copy(wgt_hbm.at[gq], wgt_v, sem)

        def drain(iset):
            idx_v, wgt_v, sem = iset
            pltpu.make_async_copy(idx_hbm.at[0], idx_v, sem).wait()
            pltpu.make_async_copy(wgt_hbm.at[0], wgt_v, sem).wait()

        def do_q(gq, iset):
            idx_v, wgt_v, _ = iset
            handles = []
            for bc in range(NBIG):
                handles.append(pltpu.async_copy(
                    v_hbm.at[idx_v.at[pl.ds(bc * BIG, BIG)]],
                    bufs[bc % DEPTH], gsems[bc % DEPTH]))
                if bc >= DEPTH - 1:
                    handles[bc - DEPTH + 1].wait()
                    process_big(wgt_v, bc - DEPTH + 1,
                                bufs[(bc - DEPTH + 1) % DEPTH])
            for bc in range(NBIG - DEPTH + 1, NBIG):
                handles[bc].wait()
                process_big(wgt_v, bc, bufs[bc % DEPTH])
            pltpu.sync_copy(out_v, out_hbm.at[gq])

        fetch(q0, isets[0])

        def per_pair(i, carry):
            qa = q0 + 2 * i
            drain(isets[0])
            fetch(qa + 1, isets[1])
            do_q(qa, isets[0])
            drain(isets[1])
            fetch(jnp.minimum(qa + 2, QP - 1), isets[0])
            do_q(qa + 1, isets[1])
            return carry

        lax.fori_loop(0, QPW // 2, per_pair, 0)
        drain(isets[0])

    return body(v_t, idx_q, wgt_q)


def _outproj_body(s_ref, m_ref, q_ref, w_ref, b_ref, o_ref):
    cnt = jnp.maximum(jnp.sum(m_ref[...], axis=1, keepdims=True), 1.0)
    s = s_ref[...] / cnt
    o_ref[...] = jnp.dot(s, w_ref[...],
                         preferred_element_type=jnp.float32) \
        + b_ref[...] + q_ref[...]


def _outproj(slots, mq, qpad, w, b):
    blk = 512
    grid = QP // blk
    return pl.pallas_call(
        _outproj_body,
        grid=(grid,),
        in_specs=[
            pl.BlockSpec((blk, EMBED), lambda i: (i, 0)),
            pl.BlockSpec((blk, CAMS), lambda i: (i, 0)),
            pl.BlockSpec((blk, EMBED), lambda i: (i, 0)),
            pl.BlockSpec((EMBED, EMBED), lambda i: (0, 0)),
            pl.BlockSpec((1, EMBED), lambda i: (0, 0)),
        ],
        out_specs=pl.BlockSpec((blk, EMBED), lambda i: (i, 0)),
        out_shape=jax.ShapeDtypeStruct((QP, EMBED), jnp.float32),
    )(slots, mq, qpad, w, b)


def kernel(query, key, value, reference_points_cam, bev_mask, spatial_shapes,
           level_start_index, W_off, b_off, W_attn, b_attn, W_val, b_val,
           W_out, b_out):
    q2 = query[0]
    v3 = value[:, :, 0, :].reshape(CAMS * L, EMBED)

    vproj = _vproj(v3, W_val, b_val.reshape(1, EMBED))
    v5 = vproj.reshape(CAMS, L, HEADS, HD).transpose(0, 2, 1, 3)
    vp = jnp.pad(v5, ((0, 0), (0, 0), (96, 96), (0, 0)))
    segs = []
    for lvl, (Hl, Wl) in enumerate(SHAPES):
        start = 96 + LSTART[lvl] - (Wl + 1)
        segs.append(jnp.concatenate(
            [vp[:, :, start + sh:start + sh + SEG[lvl], :]
             for sh in (0, 1, Wl, Wl + 1)], axis=-1))
    v4 = jnp.concatenate(segs, axis=2) \
        .reshape(CAMS * HEADS * PLANE, 4 * HD)

    pad = QP - NQ
    qT = jnp.pad(q2.T, ((0, 0), (0, pad)))
    rpc = reference_points_cam[:, 0]
    refT = jnp.pad(rpc.transpose(3, 0, 2, 1).reshape(48, NQ),
                   ((0, 0), (0, pad)))
    maskf = jnp.pad(
        bev_mask[:, 0].astype(jnp.float32).transpose(0, 2, 1).reshape(24, NQ),
        ((0, 0), (0, pad)))
    woffp = W_off.reshape(EMBED, HEADS, LEVELS, POINTS, 2) \
        .transpose(4, 2, 1, 3, 0).reshape(512, EMBED)
    boffp = b_off.reshape(HEADS, LEVELS, POINTS, 2) \
        .transpose(3, 1, 0, 2).reshape(512, 1)
    wattnT = W_attn.T
    battn = b_attn.reshape(EMBED, 1)
    base_row = jnp.broadcast_to(
        (jnp.repeat(jnp.arange(HEADS, dtype=jnp.int32), POINTS) * PLANE)
        .reshape(64, 1), (64, QB))

    idxT, wgtT, mT = _expand(qT, refT, maskf, woffp, boffp, wattnT, battn,
                             base_row)
    idx_q = idxT.reshape(CAMS, LEVELS, HEADS, 2, D, QP) \
        .transpose(5, 0, 2, 1, 3, 4).reshape(QP, IDX_PER_Q)
    wgt_q = wgtT.reshape(CAMS, LEVELS, 4, HEADS, 2, D, QP) \
        .transpose(6, 0, 3, 1, 4, 5, 2).reshape(QP, WGT_PER_Q)

    slots = _sc_gather(v4, idx_q, wgt_q)

    out = _outproj(slots, mT.T, jnp.pad(q2, ((0, pad), (0, 0))),
                   W_out, b_out.reshape(1, EMBED))
    return out[:NQ].reshape(1, NQ, EMBED)
